# baseline (speedup 1.0000x reference)
import functools

import jax
import jax.numpy as jnp
from jax import lax
from jax.experimental import pallas as pl
from jax.experimental.pallas import tpu as pltpu
from jax.experimental.pallas import tpu_sc as plsc

NC, NS, L = 2, 16, 16
NW = NC * NS
NBUF = 2


def _build_comb(segment_table, pe2):
    NSEG, D = segment_table.shape
    S = pe2.shape[0]

    def comb_tc(seg_ref, pe_ref, out_ref):
        for g in range(NSEG):
            out_ref[g * S:(g + 1) * S, :] = (
                pe_ref[...] + seg_ref[g, :][None, :])

    return pl.pallas_call(
        comb_tc,
        out_shape=jax.ShapeDtypeStruct((NSEG * S, D), jnp.float32),
    )(segment_table, pe2)


def _make_sc_kernel(B, S, D, NSEG):
    ROWS = B * S
    CHUNK = 2 * S
    RPW = ROWS // NW
    NCHUNK = RPW // CHUNK
    NITER = NCHUNK // NBUF

    mesh = plsc.VectorSubcoreMesh(
        core_axis_name="c", subcore_axis_name="s", num_cores=NC, num_subcores=NS
    )

    @functools.partial(
        pl.kernel,
        out_type=jax.ShapeDtypeStruct((ROWS, D), jnp.float32),
        mesh=mesh,
        scratch_types=[
            pltpu.VMEM((RPW,), jnp.int32),
            pltpu.VMEM((RPW,), jnp.int32),
            pltpu.VMEM((NBUF * CHUNK,), jnp.int32),
            pltpu.VMEM_SHARED((NSEG * S, D), jnp.float32),
            [pltpu.VMEM((CHUNK, D), jnp.float32) for _ in range(NBUF)],
            [pltpu.SemaphoreType.DMA for _ in range(NBUF)],
            [pltpu.SemaphoreType.DMA for _ in range(NBUF)],
        ],
    )
    def sc_kernel(ids_hbm, seg_hbm, tok_hbm, comb_hbm, out_hbm,
                  idx_all, sidx_all, crow, comb_sh, bufs, gsems, osems):
        wid = lax.axis_index("s") * NC + lax.axis_index("c")
        tbase = wid * RPW
        pltpu.sync_copy(ids_hbm.at[pl.ds(tbase, RPW)], idx_all)
        pltpu.sync_copy(seg_hbm.at[pl.ds(tbase, RPW)], sidx_all)

        @pl.when(lax.axis_index("s") == 0)
        def _():
            pltpu.sync_copy(comb_hbm, comb_sh)

        plsc.subcore_barrier()

        def iter_body(i, carry):
            c0 = i * NBUF
            for k in range(NBUF):
                c = c0 + k

                @pl.when(i > 0)
                def _():
                    pltpu.make_async_copy(
                        bufs[k], out_hbm.at[pl.ds(0, CHUNK)], osems[k]).wait()

                for jg in range(CHUNK // L):
                    j0 = jg * L
                    segv = sidx_all[pl.ds(c * CHUNK + j0, L)]
                    crow[pl.ds(k * CHUNK + j0, L)] = (
                        segv * S
                        + lax.rem(j0 + lax.iota(jnp.int32, L), S))
                pltpu.async_copy(
                    comb_sh.at[crow.at[pl.ds(k * CHUNK, CHUNK)]],
                    bufs[k], gsems[k])
            for k in range(NBUF):
                c = c0 + k
                pltpu.make_async_copy(
                    comb_sh.at[crow.at[pl.ds(k * CHUNK, CHUNK)]],
                    bufs[k], gsems[k]).wait()
                pltpu.async_copy(
                    tok_hbm.at[idx_all.at[pl.ds(c * CHUNK, CHUNK)]],
                    bufs[k], gsems[k], add=True)
            for k in range(NBUF):
                c = c0 + k
                pltpu.make_async_copy(
                    tok_hbm.at[idx_all.at[pl.ds(c * CHUNK, CHUNK)]],
                    bufs[k], gsems[k]).wait()
                pltpu.async_copy(
                    bufs[k], out_hbm.at[pl.ds(tbase + c * CHUNK, CHUNK)],
                    osems[k])
            return carry

        lax.fori_loop(0, NITER, iter_body, 0, unroll=False)
        for k in range(NBUF):
            pltpu.make_async_copy(
                bufs[k], out_hbm.at[pl.ds(0, CHUNK)], osems[k]).wait()

    return sc_kernel


def kernel(ids, segment_label, token_table, segment_table, pe):
    B, S = ids.shape
    V, D = token_table.shape
    NSEG = segment_table.shape[0]
    ids_f = ids.reshape(-1).astype(jnp.int32)
    seg_f = segment_label.reshape(-1).astype(jnp.int32)
    pe2 = pe.reshape(S, D).astype(jnp.float32)
    comb = _build_comb(segment_table.astype(jnp.float32), pe2)
    sc = _make_sc_kernel(B, S, D, NSEG)
    out = sc(ids_f, seg_f, token_table, comb)
    return out.reshape(B, S, D)


# speedup vs baseline: 1.2458x; 1.2458x over previous
import functools

import jax
import jax.numpy as jnp
from jax import lax
from jax.experimental import pallas as pl
from jax.experimental.pallas import tpu as pltpu
from jax.experimental.pallas import tpu_sc as plsc

NC, NS, L = 2, 16, 16
NW = NC * NS
NBUF = 4


def _build_comb(segment_table, pe2):
    NSEG, D = segment_table.shape
    S = pe2.shape[0]

    def comb_tc(seg_ref, pe_ref, out_ref):
        for g in range(NSEG):
            out_ref[g * S:(g + 1) * S, :] = (
                pe_ref[...] + seg_ref[g, :][None, :])

    return pl.pallas_call(
        comb_tc,
        out_shape=jax.ShapeDtypeStruct((NSEG * S, D), jnp.float32),
    )(segment_table, pe2)


def _make_sc_kernel(B, S, D, NSEG):
    ROWS = B * S
    CHUNK = S
    RPW = ROWS // NW
    NCHUNK = RPW // CHUNK
    NITER = NCHUNK // NBUF

    mesh = plsc.VectorSubcoreMesh(
        core_axis_name="c", subcore_axis_name="s", num_cores=NC, num_subcores=NS
    )

    @functools.partial(
        pl.kernel,
        out_type=jax.ShapeDtypeStruct((ROWS, D), jnp.float32),
        mesh=mesh,
        scratch_types=[
            pltpu.VMEM((RPW,), jnp.int32),
            pltpu.VMEM((RPW,), jnp.int32),
            pltpu.VMEM((NBUF * CHUNK,), jnp.int32),
            pltpu.VMEM_SHARED((NSEG * S, D), jnp.float32),
            [pltpu.VMEM((CHUNK, D), jnp.float32) for _ in range(NBUF)],
            [pltpu.SemaphoreType.DMA for _ in range(NBUF)],
            [pltpu.SemaphoreType.DMA for _ in range(NBUF)],
        ],
    )
    def sc_kernel(ids_hbm, seg_hbm, tok_hbm, comb_hbm, out_hbm,
                  idx_all, sidx_all, crow, comb_sh, bufs, gsems, osems):
        wid = lax.axis_index("s") * NC + lax.axis_index("c")
        tbase = wid * RPW
        pltpu.sync_copy(ids_hbm.at[pl.ds(tbase, RPW)], idx_all)
        pltpu.sync_copy(seg_hbm.at[pl.ds(tbase, RPW)], sidx_all)

        @pl.when(lax.axis_index("s") == 0)
        def _():
            pltpu.sync_copy(comb_hbm, comb_sh)

        plsc.subcore_barrier()

        def iter_body(i, carry):
            c0 = i * NBUF
            for k in range(NBUF):
                c = c0 + k

                @pl.when(i > 0)
                def _():
                    pltpu.make_async_copy(
                        bufs[k], out_hbm.at[pl.ds(0, CHUNK)], osems[k]).wait()

                for jg in range(CHUNK // L):
                    j0 = jg * L
                    segv = sidx_all[pl.ds(c * CHUNK + j0, L)]
                    crow[pl.ds(k * CHUNK + j0, L)] = (
                        segv * S + (j0 + lax.iota(jnp.int32, L)))
                pltpu.async_copy(
                    comb_sh.at[crow.at[pl.ds(k * CHUNK, CHUNK)]],
                    bufs[k], gsems[k])
            for k in range(NBUF):
                c = c0 + k
                pltpu.make_async_copy(
                    comb_sh.at[crow.at[pl.ds(k * CHUNK, CHUNK)]],
                    bufs[k], gsems[k]).wait()
                pltpu.async_copy(
                    tok_hbm.at[idx_all.at[pl.ds(c * CHUNK, CHUNK)]],
                    bufs[k], gsems[k], add=True)
            for k in range(NBUF):
                c = c0 + k
                pltpu.make_async_copy(
                    tok_hbm.at[idx_all.at[pl.ds(c * CHUNK, CHUNK)]],
                    bufs[k], gsems[k]).wait()
                pltpu.async_copy(
                    bufs[k], out_hbm.at[pl.ds(tbase + c * CHUNK, CHUNK)],
                    osems[k])
            return carry

        lax.fori_loop(0, NITER, iter_body, 0, unroll=False)
        for k in range(NBUF):
            pltpu.make_async_copy(
                bufs[k], out_hbm.at[pl.ds(0, CHUNK)], osems[k]).wait()

    return sc_kernel


def kernel(ids, segment_label, token_table, segment_table, pe):
    B, S = ids.shape
    V, D = token_table.shape
    NSEG = segment_table.shape[0]
    ids_f = ids.reshape(-1).astype(jnp.int32)
    seg_f = segment_label.reshape(-1).astype(jnp.int32)
    pe2 = pe.reshape(S, D).astype(jnp.float32)
    comb = _build_comb(segment_table.astype(jnp.float32), pe2)
    sc = _make_sc_kernel(B, S, D, NSEG)
    out = sc(ids_f, seg_f, token_table, comb)
    return out.reshape(B, S, D)
